# Initial kernel scaffold; baseline (speedup 1.0000x reference)
import jax
import jax.numpy as jnp
from jax.experimental import pallas as pl


def kernel(X, node_idx, hedge_idx, W, b):
    raise NotImplementedError("write your pallas kernel here")


# speedup vs baseline: 5.0591x; 5.0591x over previous
import functools

import jax
import jax.numpy as jnp
from jax import lax
from jax.experimental import pallas as pl
from jax.experimental.pallas import tpu as pltpu
from jax.experimental.pallas import tpu_sc as plsc

NC, NS = 2, 16
NW = NC * NS
C = 80
D = 128

NV = 10000
NE = 5000
NV_PAD = 10240
NE_PAD = 5120


def _mesh():
  return plsc.VectorSubcoreMesh(
      core_axis_name="c", subcore_axis_name="s", num_cores=NC, num_subcores=NS)


def _zero_1d(ref, n):
  def body(i, _):
    ref[pl.ds(i * 16, 16)] = jnp.zeros((16,), jnp.float32)
    return 0
  lax.fori_loop(0, n // 16, body, 0)


def _zero_2d(ref, nrows, ncols):
  k = ncols // 16
  def body(i, _):
    r = i // k
    j = i % k
    ref[r, pl.ds(j * 16, 16)] = jnp.zeros((16,), jnp.float32)
    return 0
  lax.fori_loop(0, nrows * k, body, 0)


@functools.partial(
    pl.kernel,
    out_type=(jax.ShapeDtypeStruct((NC * NV_PAD,), jnp.float32),
              jax.ShapeDtypeStruct((NC * NE_PAD,), jnp.float32)),
    mesh=_mesh(),
    scratch_types=[
        pltpu.VMEM((C,), jnp.int32),
        pltpu.VMEM((C,), jnp.float32),
        pltpu.VMEM((NV_PAD // NS,), jnp.float32),
        pltpu.VMEM_SHARED((NV_PAD,), jnp.float32),
        pltpu.VMEM_SHARED((NE_PAD,), jnp.float32),
    ],
)
def _hist(nidx, hidx, dv_out, de_out, idx_v, ones_v, zeros_v, dv_s, de_s):
  c = lax.axis_index("c")
  s = lax.axis_index("s")
  wid = s * NC + c
  per_w = nidx.shape[0] // NW
  nchunks = per_w // C

  _zero_1d(zeros_v, NV_PAD // NS)

  def ob(i, _):
    ones_v[pl.ds(i * 16, 16)] = jnp.ones((16,), jnp.float32)
    return 0
  lax.fori_loop(0, C // 16, ob, 0)

  vslice = NV_PAD // NS
  eslice = NE_PAD // NS
  pltpu.sync_copy(zeros_v, dv_s.at[pl.ds(s * vslice, vslice)])
  pltpu.sync_copy(zeros_v.at[pl.ds(0, eslice)], de_s.at[pl.ds(s * eslice, eslice)])
  plsc.subcore_barrier()

  def body(j, _):
    base = wid * per_w + j * C
    pltpu.sync_copy(nidx.at[pl.ds(base, C)], idx_v)
    pltpu.sync_copy(ones_v, dv_s.at[idx_v], add=True)
    pltpu.sync_copy(hidx.at[pl.ds(base, C)], idx_v)
    pltpu.sync_copy(ones_v, de_s.at[idx_v], add=True)
    return 0
  lax.fori_loop(0, nchunks, body, 0)
  plsc.subcore_barrier()

  pltpu.sync_copy(dv_s.at[pl.ds(s * vslice, vslice)], zeros_v)
  pltpu.sync_copy(zeros_v, dv_out.at[pl.ds(c * NV_PAD + s * vslice, vslice)])
  pltpu.sync_copy(de_s.at[pl.ds(s * eslice, eslice)],
                  zeros_v.at[pl.ds(0, eslice)])
  pltpu.sync_copy(zeros_v.at[pl.ds(0, eslice)],
                  de_out.at[pl.ds(c * NE_PAD + s * eslice, eslice)])


def _proj_body(x_ref, w_ref, b_ref, dvp_ref, dep_ref, hs_ref, dvi_ref, dei_ref):
  dv = dvp_ref[0] + dvp_ref[1]
  dvi = jnp.where(dv > 0, lax.rsqrt(dv), 0.0)
  de = dep_ref[0] + dep_ref[1]
  dei = jnp.where(de > 0, 1.0 / de, 0.0)
  h = lax.dot_general(x_ref[...], w_ref[...], (((1,), (1,)), ((), ())),
                      preferred_element_type=jnp.float32,
                      precision=lax.Precision.HIGHEST)
  hs_ref[...] = (h + b_ref[...]) * dvi
  dvi_ref[...] = dvi
  dei_ref[...] = dei


def _proj(x, w, b2, dvp, dep):
  return pl.pallas_call(
      _proj_body,
      out_shape=(jax.ShapeDtypeStruct((NV, D), jnp.float32),
                 jax.ShapeDtypeStruct((NV, 1), jnp.float32),
                 jax.ShapeDtypeStruct((NE, 1), jnp.float32)),
  )(x, w, b2, dvp, dep)


@functools.partial(
    pl.kernel,
    out_type=jax.ShapeDtypeStruct((NC, NE_PAD, D), jnp.float32),
    mesh=_mesh(),
    scratch_types=[
        pltpu.VMEM((C,), jnp.int32),
        pltpu.VMEM((C,), jnp.int32),
        pltpu.VMEM((C, D), jnp.float32),
        pltpu.VMEM_SHARED((NE_PAD, D), jnp.float32),
        pltpu.SemaphoreType.DMA,
    ],
)
def _hedge_agg(hs, nidx, hidx, ye_out, nidx_v, hidx_v, rows_v, ye_s, sem):
  c = lax.axis_index("c")
  s = lax.axis_index("s")
  wid = s * NC + c
  per_w = nidx.shape[0] // NW
  nchunks = per_w // C

  _zero_2d(rows_v, C, D)
  rows_per_tile = NE_PAD // NS

  def zb(i, _):
    pltpu.sync_copy(rows_v, ye_s.at[pl.ds(s * rows_per_tile + i * C, C)])
    return 0
  lax.fori_loop(0, rows_per_tile // C, zb, 0)
  plsc.subcore_barrier()

  def body(j, _):
    base = wid * per_w + j * C
    pltpu.sync_copy(nidx.at[pl.ds(base, C)], nidx_v)
    pltpu.sync_copy(hidx.at[pl.ds(base, C)], hidx_v)
    pltpu.async_copy(hs.at[nidx_v], rows_v, sem).wait()
    pltpu.sync_copy(rows_v, ye_s.at[hidx_v], add=True)
    return 0
  lax.fori_loop(0, nchunks, body, 0)
  plsc.subcore_barrier()

  def db(i, _):
    off = s * rows_per_tile + i * C
    pltpu.sync_copy(ye_s.at[pl.ds(off, C)], rows_v)
    pltpu.sync_copy(rows_v, ye_out.at[c, pl.ds(off, C)])
    return 0
  lax.fori_loop(0, rows_per_tile // C, db, 0)


def _ye_combine_body(yep_ref, dei_ref, ye_ref):
  ye = yep_ref[0, :NE, :] + yep_ref[1, :NE, :]
  ye_ref[...] = ye * dei_ref[...]


def _ye_combine(yep, dei):
  return pl.pallas_call(
      _ye_combine_body,
      out_shape=jax.ShapeDtypeStruct((NE, D), jnp.float32),
  )(yep, dei)


@functools.partial(
    pl.kernel,
    out_type=jax.ShapeDtypeStruct((NC, NV_PAD, D), jnp.float32),
    mesh=_mesh(),
    scratch_types=[
        pltpu.VMEM((C,), jnp.int32),
        pltpu.VMEM((C,), jnp.int32),
        pltpu.VMEM((C, D), jnp.float32),
        pltpu.VMEM_SHARED((NV_PAD, D), jnp.float32),
        pltpu.SemaphoreType.DMA,
    ],
)
def _node_agg(ye, nidx, hidx, z_out, nidx_v, hidx_v, rows_v, z_s, sem):
  c = lax.axis_index("c")
  s = lax.axis_index("s")
  wid = s * NC + c
  per_w = nidx.shape[0] // NW
  nchunks = per_w // C

  _zero_2d(rows_v, C, D)
  rows_per_tile = NV_PAD // NS

  def zb(i, _):
    pltpu.sync_copy(rows_v, z_s.at[pl.ds(s * rows_per_tile + i * C, C)])
    return 0
  lax.fori_loop(0, rows_per_tile // C, zb, 0)
  plsc.subcore_barrier()

  def body(j, _):
    base = wid * per_w + j * C
    pltpu.sync_copy(nidx.at[pl.ds(base, C)], nidx_v)
    pltpu.sync_copy(hidx.at[pl.ds(base, C)], hidx_v)
    pltpu.async_copy(ye.at[hidx_v], rows_v, sem).wait()
    pltpu.sync_copy(rows_v, z_s.at[nidx_v], add=True)
    return 0
  lax.fori_loop(0, nchunks, body, 0)
  plsc.subcore_barrier()

  def db(i, _):
    off = s * rows_per_tile + i * C
    pltpu.sync_copy(z_s.at[pl.ds(off, C)], rows_v)
    pltpu.sync_copy(rows_v, z_out.at[c, pl.ds(off, C)])
    return 0
  lax.fori_loop(0, rows_per_tile // C, db, 0)


def _z_combine_body(zp_ref, dvi_ref, z_ref):
  z = zp_ref[0, :NV, :] + zp_ref[1, :NV, :]
  z_ref[...] = jnp.maximum(z * dvi_ref[...], 0.0)


def _z_combine(zp, dvi):
  return pl.pallas_call(
      _z_combine_body,
      out_shape=jax.ShapeDtypeStruct((NV, D), jnp.float32),
  )(zp, dvi)


def kernel(X, node_idx, hedge_idx, W, b):
  nidx = node_idx.astype(jnp.int32)
  hidx = hedge_idx.astype(jnp.int32)
  dvp, dep = _hist(nidx, hidx)
  dvp = dvp.reshape(NC, NV_PAD)[:, :NV, None]
  dep = dep.reshape(NC, NE_PAD)[:, :NE, None]
  hs, dvi, dei = _proj(X, W, b.reshape(1, D), dvp, dep)
  yep = _hedge_agg(hs, nidx, hidx)
  ye = _ye_combine(yep, dei)
  zp = _node_agg(ye, nidx, hidx)
  return _z_combine(zp, dvi)


# speedup vs baseline: 7.6646x; 1.5150x over previous
import functools

import jax
import jax.numpy as jnp
from jax import lax
from jax.experimental import pallas as pl
from jax.experimental.pallas import tpu as pltpu
from jax.experimental.pallas import tpu_sc as plsc

NC, NS = 2, 16
NW = NC * NS
C = 80
D = 128

NV = 10000
NE = 5000
NV_PAD = 10240
NE_PAD = 5120


def _mesh():
  return plsc.VectorSubcoreMesh(
      core_axis_name="c", subcore_axis_name="s", num_cores=NC, num_subcores=NS)


def _zero_1d(ref, n):
  def body(i, _):
    ref[pl.ds(i * 16, 16)] = jnp.zeros((16,), jnp.float32)
    return 0
  lax.fori_loop(0, n // 16, body, 0)


def _zero_2d(ref, nrows, ncols):
  k = ncols // 16
  def body(i, _):
    r = i // k
    j = i % k
    ref[r, pl.ds(j * 16, 16)] = jnp.zeros((16,), jnp.float32)
    return 0
  lax.fori_loop(0, nrows * k, body, 0)


@functools.partial(
    pl.kernel,
    out_type=(jax.ShapeDtypeStruct((NC * NV_PAD,), jnp.float32),
              jax.ShapeDtypeStruct((NC * NE_PAD,), jnp.float32)),
    mesh=_mesh(),
    scratch_types=[
        pltpu.VMEM((C,), jnp.int32),
        pltpu.VMEM((C,), jnp.int32),
        pltpu.VMEM((C,), jnp.int32),
        pltpu.VMEM((C,), jnp.int32),
        pltpu.VMEM((C,), jnp.float32),
        pltpu.VMEM((NV_PAD // NS,), jnp.float32),
        pltpu.VMEM_SHARED((NV_PAD,), jnp.float32),
        pltpu.VMEM_SHARED((NE_PAD,), jnp.float32),
        pltpu.SemaphoreType.DMA,
        pltpu.SemaphoreType.DMA,
        pltpu.SemaphoreType.DMA,
        pltpu.SemaphoreType.DMA,
    ],
)
def _hist(nidx, hidx, dv_out, de_out,
          in0, in1, ih0, ih1, ones_v, zeros_v, dv_s, de_s,
          sn0, sn1, sh0, sh1):
  c = lax.axis_index("c")
  s = lax.axis_index("s")
  wid = s * NC + c
  per_w = nidx.shape[0] // NW
  nchunks = per_w // C

  _zero_1d(zeros_v, NV_PAD // NS)

  def ob(i, _):
    ones_v[pl.ds(i * 16, 16)] = jnp.ones((16,), jnp.float32)
    return 0
  lax.fori_loop(0, C // 16, ob, 0)

  vslice = NV_PAD // NS
  eslice = NE_PAD // NS
  pltpu.sync_copy(zeros_v, dv_s.at[pl.ds(s * vslice, vslice)])
  pltpu.sync_copy(zeros_v.at[pl.ds(0, eslice)], de_s.at[pl.ds(s * eslice, eslice)])
  plsc.subcore_barrier()

  def slot_step(j, me_in, me_ih, me_sn, me_sh):
    @pl.when(j >= 2)
    def _():
      pltpu.make_async_copy(ones_v, dv_s.at[me_in], me_sn).wait()
      pltpu.make_async_copy(ones_v, de_s.at[me_ih], me_sh).wait()
    base = wid * per_w + j * C
    pltpu.sync_copy(nidx.at[pl.ds(base, C)], me_in)
    pltpu.sync_copy(hidx.at[pl.ds(base, C)], me_ih)
    pltpu.async_copy(ones_v, dv_s.at[me_in], me_sn, add=True)
    pltpu.async_copy(ones_v, de_s.at[me_ih], me_sh, add=True)

  def body(j, _):
    @pl.when(j % 2 == 0)
    def _():
      slot_step(j, in0, ih0, sn0, sh0)
    @pl.when(j % 2 == 1)
    def _():
      slot_step(j, in1, ih1, sn1, sh1)
    return 0
  lax.fori_loop(0, nchunks, body, 0)

  pltpu.make_async_copy(ones_v, dv_s.at[in0], sn0).wait()
  pltpu.make_async_copy(ones_v, de_s.at[ih0], sh0).wait()
  pltpu.make_async_copy(ones_v, dv_s.at[in1], sn1).wait()
  pltpu.make_async_copy(ones_v, de_s.at[ih1], sh1).wait()
  plsc.subcore_barrier()

  pltpu.sync_copy(dv_s.at[pl.ds(s * vslice, vslice)], zeros_v)
  pltpu.sync_copy(zeros_v, dv_out.at[pl.ds(c * NV_PAD + s * vslice, vslice)])
  pltpu.sync_copy(de_s.at[pl.ds(s * eslice, eslice)],
                  zeros_v.at[pl.ds(0, eslice)])
  pltpu.sync_copy(zeros_v.at[pl.ds(0, eslice)],
                  de_out.at[pl.ds(c * NE_PAD + s * eslice, eslice)])


def _proj_body(x_ref, w_ref, b_ref, dvp_ref, dep_ref, hs_ref, dvi_ref, dei_ref):
  dv = dvp_ref[0] + dvp_ref[1]
  dvi = jnp.where(dv > 0, lax.rsqrt(dv), 0.0)
  de = dep_ref[0] + dep_ref[1]
  dei = jnp.where(de > 0, 1.0 / de, 0.0)
  h = lax.dot_general(x_ref[...], w_ref[...], (((1,), (1,)), ((), ())),
                      preferred_element_type=jnp.float32,
                      precision=lax.Precision.HIGHEST)
  hs_ref[...] = (h + b_ref[...]) * dvi
  dvi_ref[...] = dvi
  dei_ref[...] = dei


def _proj(x, w, b2, dvp, dep):
  return pl.pallas_call(
      _proj_body,
      out_shape=(jax.ShapeDtypeStruct((NV, D), jnp.float32),
                 jax.ShapeDtypeStruct((NV, 1), jnp.float32),
                 jax.ShapeDtypeStruct((NE, 1), jnp.float32)),
  )(x, w, b2, dvp, dep)


def _make_agg(acc_rows):
  @functools.partial(
      pl.kernel,
      out_type=jax.ShapeDtypeStruct((NC, acc_rows, D), jnp.float32),
      mesh=_mesh(),
      scratch_types=[
          pltpu.VMEM((C,), jnp.int32),
          pltpu.VMEM((C,), jnp.int32),
          pltpu.VMEM((C,), jnp.int32),
          pltpu.VMEM((C,), jnp.int32),
          pltpu.VMEM((C, D), jnp.float32),
          pltpu.VMEM((C, D), jnp.float32),
          pltpu.VMEM_SHARED((acc_rows, D), jnp.float32),
          pltpu.SemaphoreType.DMA,
          pltpu.SemaphoreType.DMA,
          pltpu.SemaphoreType.DMA,
          pltpu.SemaphoreType.DMA,
      ],
  )
  def agg(table, gidx, sidx, out,
          g0, g1, s0, s1, rows0, rows1, acc_s, gs0, gs1, ss0, ss1):
    c = lax.axis_index("c")
    s = lax.axis_index("s")
    wid = s * NC + c
    per_w = gidx.shape[0] // NW
    nchunks = per_w // C

    _zero_2d(rows0, C, D)
    rows_per_tile = acc_rows // NS

    def zb(i, _):
      pltpu.sync_copy(rows0, acc_s.at[pl.ds(s * rows_per_tile + i * C, C)])
      return 0
    lax.fori_loop(0, rows_per_tile // C, zb, 0)
    plsc.subcore_barrier()

    base0 = wid * per_w
    pltpu.sync_copy(gidx.at[pl.ds(base0, C)], g0)
    pltpu.sync_copy(sidx.at[pl.ds(base0, C)], s0)
    pltpu.async_copy(table.at[g0], rows0, gs0)

    def slot_step(j, me_g, me_s, me_rows, me_gs, me_ss,
                  ot_g, ot_s, ot_rows, ot_gs, ot_ss):
      @pl.when(j + 1 < nchunks)
      def _():
        @pl.when(j >= 1)
        def _():
          pltpu.make_async_copy(ot_rows, acc_s.at[ot_s], ot_ss).wait()
        nbase = wid * per_w + (j + 1) * C
        pltpu.sync_copy(gidx.at[pl.ds(nbase, C)], ot_g)
        pltpu.sync_copy(sidx.at[pl.ds(nbase, C)], ot_s)
        pltpu.async_copy(table.at[ot_g], ot_rows, ot_gs)
      pltpu.make_async_copy(table.at[me_g], me_rows, me_gs).wait()
      pltpu.async_copy(me_rows, acc_s.at[me_s], me_ss, add=True)

    def body(j, _):
      @pl.when(j % 2 == 0)
      def _():
        slot_step(j, g0, s0, rows0, gs0, ss0, g1, s1, rows1, gs1, ss1)
      @pl.when(j % 2 == 1)
      def _():
        slot_step(j, g1, s1, rows1, gs1, ss1, g0, s0, rows0, gs0, ss0)
      return 0
    lax.fori_loop(0, nchunks, body, 0)

    pltpu.make_async_copy(rows0, acc_s.at[s0], ss0).wait()
    pltpu.make_async_copy(rows1, acc_s.at[s1], ss1).wait()
    plsc.subcore_barrier()

    def db(i, _):
      off = s * rows_per_tile + i * C
      pltpu.sync_copy(acc_s.at[pl.ds(off, C)], rows0)
      pltpu.sync_copy(rows0, out.at[c, pl.ds(off, C)])
      return 0
    lax.fori_loop(0, rows_per_tile // C, db, 0)

  return agg


_hedge_agg = _make_agg(NE_PAD)
_node_agg = _make_agg(NV_PAD)


def _ye_combine_body(yep_ref, dei_ref, ye_ref):
  ye = yep_ref[0, :NE, :] + yep_ref[1, :NE, :]
  ye_ref[...] = ye * dei_ref[...]


def _ye_combine(yep, dei):
  return pl.pallas_call(
      _ye_combine_body,
      out_shape=jax.ShapeDtypeStruct((NE, D), jnp.float32),
  )(yep, dei)


def _z_combine_body(zp_ref, dvi_ref, z_ref):
  z = zp_ref[0, :NV, :] + zp_ref[1, :NV, :]
  z_ref[...] = jnp.maximum(z * dvi_ref[...], 0.0)


def _z_combine(zp, dvi):
  return pl.pallas_call(
      _z_combine_body,
      out_shape=jax.ShapeDtypeStruct((NV, D), jnp.float32),
  )(zp, dvi)


def kernel(X, node_idx, hedge_idx, W, b):
  nidx = node_idx.astype(jnp.int32)
  hidx = hedge_idx.astype(jnp.int32)
  dvp, dep = _hist(nidx, hidx)
  dvp = dvp.reshape(NC, NV_PAD)[:, :NV, None]
  dep = dep.reshape(NC, NE_PAD)[:, :NE, None]
  hs, dvi, dei = _proj(X, W, b.reshape(1, D), dvp, dep)
  yep = _hedge_agg(hs, nidx, hidx)
  ye = _ye_combine(yep, dei)
  zp = _node_agg(ye, hidx, nidx)
  return _z_combine(zp, dvi)


# speedup vs baseline: 10.0704x; 1.3139x over previous
import functools

import jax
import jax.numpy as jnp
from jax import lax
from jax.experimental import pallas as pl
from jax.experimental.pallas import tpu as pltpu
from jax.experimental.pallas import tpu_sc as plsc

NC, NS = 2, 16
NW = NC * NS
C = 80
D = 128

NV = 10000
NE = 5000
NV_PAD = 10240
NE_PAD = 5120
NNZ = 320000
PW = NNZ // NW
NCH = PW // C


def _mesh():
  return plsc.VectorSubcoreMesh(
      core_axis_name="c", subcore_axis_name="s", num_cores=NC, num_subcores=NS)


def _zero_1d(ref, n):
  def body(i, _):
    ref[pl.ds(i * 16, 16)] = jnp.zeros((16,), jnp.float32)
    return 0
  lax.fori_loop(0, n // 16, body, 0)


def _zero_2d(ref, nrows, ncols):
  k = ncols // 16
  def body(i, _):
    r = i // k
    j = i % k
    ref[r, pl.ds(j * 16, 16)] = jnp.zeros((16,), jnp.float32)
    return 0
  lax.fori_loop(0, nrows * k, body, 0)


@functools.partial(
    pl.kernel,
    out_type=(jax.ShapeDtypeStruct((NC * NV_PAD,), jnp.float32),
              jax.ShapeDtypeStruct((NC * NE_PAD,), jnp.float32)),
    mesh=_mesh(),
    scratch_types=[
        pltpu.VMEM((C,), jnp.int32),
        pltpu.VMEM((C,), jnp.int32),
        pltpu.VMEM((C,), jnp.int32),
        pltpu.VMEM((C,), jnp.int32),
        pltpu.VMEM((C,), jnp.float32),
        pltpu.VMEM((NV_PAD // NS,), jnp.float32),
        pltpu.VMEM_SHARED((NV_PAD,), jnp.float32),
        pltpu.VMEM_SHARED((NE_PAD,), jnp.float32),
        pltpu.SemaphoreType.DMA,
        pltpu.SemaphoreType.DMA,
        pltpu.SemaphoreType.DMA,
        pltpu.SemaphoreType.DMA,
    ],
)
def _hist(nidx, hidx, dv_out, de_out,
          in0, in1, ih0, ih1, ones_v, zeros_v, dv_s, de_s,
          sn0, sn1, sh0, sh1):
  c = lax.axis_index("c")
  s = lax.axis_index("s")
  wid = s * NC + c

  _zero_1d(zeros_v, NV_PAD // NS)

  def ob(i, _):
    ones_v[pl.ds(i * 16, 16)] = jnp.ones((16,), jnp.float32)
    return 0
  lax.fori_loop(0, C // 16, ob, 0)

  vslice = NV_PAD // NS
  eslice = NE_PAD // NS
  pltpu.sync_copy(zeros_v, dv_s.at[pl.ds(s * vslice, vslice)])
  pltpu.sync_copy(zeros_v.at[pl.ds(0, eslice)], de_s.at[pl.ds(s * eslice, eslice)])
  plsc.subcore_barrier()

  def slot_step(j, me_in, me_ih, me_sn, me_sh):
    @pl.when(j >= 2)
    def _():
      pltpu.make_async_copy(ones_v, dv_s.at[me_in], me_sn).wait()
      pltpu.make_async_copy(ones_v, de_s.at[me_ih], me_sh).wait()
    base = wid * PW + j * C
    pltpu.sync_copy(nidx.at[pl.ds(base, C)], me_in)
    pltpu.sync_copy(hidx.at[pl.ds(base, C)], me_ih)
    pltpu.async_copy(ones_v, dv_s.at[me_in], me_sn, add=True)
    pltpu.async_copy(ones_v, de_s.at[me_ih], me_sh, add=True)

  def body(j, _):
    @pl.when(j % 2 == 0)
    def _():
      slot_step(j, in0, ih0, sn0, sh0)
    @pl.when(j % 2 == 1)
    def _():
      slot_step(j, in1, ih1, sn1, sh1)
    return 0
  lax.fori_loop(0, NCH, body, 0)

  pltpu.make_async_copy(ones_v, dv_s.at[in0], sn0).wait()
  pltpu.make_async_copy(ones_v, de_s.at[ih0], sh0).wait()
  pltpu.make_async_copy(ones_v, dv_s.at[in1], sn1).wait()
  pltpu.make_async_copy(ones_v, de_s.at[ih1], sh1).wait()
  plsc.subcore_barrier()

  pltpu.sync_copy(dv_s.at[pl.ds(s * vslice, vslice)], zeros_v)
  pltpu.sync_copy(zeros_v, dv_out.at[pl.ds(c * NV_PAD + s * vslice, vslice)])
  pltpu.sync_copy(de_s.at[pl.ds(s * eslice, eslice)],
                  zeros_v.at[pl.ds(0, eslice)])
  pltpu.sync_copy(zeros_v.at[pl.ds(0, eslice)],
                  de_out.at[pl.ds(c * NE_PAD + s * eslice, eslice)])


def _proj_body(x_ref, w_ref, b_ref, dvp_ref, dep_ref, hs_ref, dvi_ref, dei_ref):
  dv = dvp_ref[0] + dvp_ref[1]
  dvi = jnp.where(dv > 0, lax.rsqrt(dv), 0.0)
  de = dep_ref[0] + dep_ref[1]
  dei = jnp.where(de > 0, 1.0 / de, 0.0)
  h = lax.dot_general(x_ref[...], w_ref[...], (((1,), (1,)), ((), ())),
                      preferred_element_type=jnp.float32,
                      precision=lax.Precision.HIGHEST)
  hs_ref[...] = (h + b_ref[...]) * dvi
  dvi_ref[...] = dvi
  dei_ref[...] = dei


def _proj(x, w, b2, dvp, dep):
  return pl.pallas_call(
      _proj_body,
      out_shape=(jax.ShapeDtypeStruct((NV, D), jnp.float32),
                 jax.ShapeDtypeStruct((NV, 1), jnp.float32),
                 jax.ShapeDtypeStruct((NE, 1), jnp.float32)),
  )(x, w, b2, dvp, dep)


def _make_agg(acc_rows):
  @functools.partial(
      pl.kernel,
      out_type=jax.ShapeDtypeStruct((NC, acc_rows, D), jnp.float32),
      mesh=_mesh(),
      scratch_types=[
          pltpu.VMEM((PW,), jnp.int32),
          pltpu.VMEM((PW,), jnp.int32),
          pltpu.VMEM((C,), jnp.int32),
          pltpu.VMEM((C,), jnp.int32),
          pltpu.VMEM((C,), jnp.int32),
          pltpu.VMEM((C,), jnp.int32),
          pltpu.VMEM((C, D), jnp.float32),
          pltpu.VMEM((C, D), jnp.float32),
          pltpu.VMEM_SHARED((acc_rows, D), jnp.float32),
          pltpu.SemaphoreType.DMA,
          pltpu.SemaphoreType.DMA,
          pltpu.SemaphoreType.DMA,
          pltpu.SemaphoreType.DMA,
      ],
  )
  def agg(table, gidx, sidx, out,
          gl, sl, g0, g1, s0, s1, rows0, rows1, acc_s, gs0, gs1, ss0, ss1):
    c = lax.axis_index("c")
    s = lax.axis_index("s")
    wid = s * NC + c

    def fill(dst, src, j):
      def fb(k, _):
        dst[pl.ds(k * 16, 16)] = src[pl.ds(j * C + k * 16, 16)]
        return 0
      lax.fori_loop(0, C // 16, fb, 0)

    pltpu.sync_copy(gidx.at[pl.ds(wid * PW, PW)], gl)
    pltpu.sync_copy(sidx.at[pl.ds(wid * PW, PW)], sl)
    fill(g0, gl, 0)
    fill(s0, sl, 0)
    pltpu.async_copy(table.at[g0], rows0, gs0)

    _zero_2d(rows1, C, D)
    rows_per_tile = acc_rows // NS

    def zb(i, _):
      pltpu.sync_copy(rows1, acc_s.at[pl.ds(s * rows_per_tile + i * C, C)])
      return 0
    lax.fori_loop(0, rows_per_tile // C, zb, 0)
    plsc.subcore_barrier()

    def slot_step(j, me_rows, me_gs, me_ss, me_g, me_s,
                  ot_rows, ot_gs, ot_ss, ot_g, ot_s):
      @pl.when(j + 1 < NCH)
      def _():
        @pl.when(j >= 1)
        def _():
          pltpu.make_async_copy(ot_rows, acc_s.at[ot_s], ot_ss).wait()
        fill(ot_g, gl, j + 1)
        fill(ot_s, sl, j + 1)
        pltpu.async_copy(table.at[ot_g], ot_rows, ot_gs)
      pltpu.make_async_copy(table.at[me_g], me_rows, me_gs).wait()
      pltpu.async_copy(me_rows, acc_s.at[me_s], me_ss, add=True)

    def body(j, _):
      @pl.when(j % 2 == 0)
      def _():
        slot_step(j, rows0, gs0, ss0, g0, s0, rows1, gs1, ss1, g1, s1)
      @pl.when(j % 2 == 1)
      def _():
        slot_step(j, rows1, gs1, ss1, g1, s1, rows0, gs0, ss0, g0, s0)
      return 0
    lax.fori_loop(0, NCH, body, 0)

    pltpu.make_async_copy(rows0, acc_s.at[s0], ss0).wait()
    pltpu.make_async_copy(rows1, acc_s.at[s1], ss1).wait()
    plsc.subcore_barrier()

    def db(i, _):
      off = s * rows_per_tile + i * C
      pltpu.sync_copy(acc_s.at[pl.ds(off, C)], rows0)
      pltpu.sync_copy(rows0, out.at[c, pl.ds(off, C)])
      return 0
    lax.fori_loop(0, rows_per_tile // C, db, 0)

  return agg


_hedge_agg = _make_agg(NE_PAD)
_node_agg = _make_agg(NV_PAD)


def _ye_combine_body(yep_ref, dei_ref, ye_ref):
  ye = yep_ref[0, :NE, :] + yep_ref[1, :NE, :]
  ye_ref[...] = ye * dei_ref[...]


def _ye_combine(yep, dei):
  return pl.pallas_call(
      _ye_combine_body,
      out_shape=jax.ShapeDtypeStruct((NE, D), jnp.float32),
  )(yep, dei)


def _z_combine_body(zp_ref, dvi_ref, z_ref):
  z = zp_ref[0, :NV, :] + zp_ref[1, :NV, :]
  z_ref[...] = jnp.maximum(z * dvi_ref[...], 0.0)


def _z_combine(zp, dvi):
  return pl.pallas_call(
      _z_combine_body,
      out_shape=jax.ShapeDtypeStruct((NV, D), jnp.float32),
  )(zp, dvi)


def kernel(X, node_idx, hedge_idx, W, b):
  nidx = node_idx.astype(jnp.int32)
  hidx = hedge_idx.astype(jnp.int32)
  dvp, dep = _hist(nidx, hidx)
  dvp = dvp.reshape(NC, NV_PAD)[:, :NV, None]
  dep = dep.reshape(NC, NE_PAD)[:, :NE, None]
  hs, dvi, dei = _proj(X, W, b.reshape(1, D), dvp, dep)
  yep = _hedge_agg(hs, nidx, hidx)
  ye = _ye_combine(yep, dei)
  zp = _node_agg(ye, hidx, nidx)
  return _z_combine(zp, dvi)


# speedup vs baseline: 13.3111x; 1.3218x over previous
import functools

import jax
import jax.numpy as jnp
from jax import lax
from jax.experimental import pallas as pl
from jax.experimental.pallas import tpu as pltpu
from jax.experimental.pallas import tpu_sc as plsc

NC, NS = 2, 16
NW = NC * NS
C = 80
D = 128

NV = 10000
NE = 5000
NV_PAD = 10240
NE_PAD = 5120
NNZ = 320000
PW = NNZ // NW
NCH = PW // C
SLAB = PW


def _mesh():
  return plsc.VectorSubcoreMesh(
      core_axis_name="c", subcore_axis_name="s", num_cores=NC, num_subcores=NS)


def _zero_1d(ref, n):
  def body(i, _):
    ref[pl.ds(i * 16, 16)] = jnp.zeros((16,), jnp.float32)
    return 0
  lax.fori_loop(0, n // 16, body, 0)


def _zero_2d(ref, nrows, ncols):
  k = ncols // 16
  def body(i, _):
    r = i // k
    j = i % k
    ref[r, pl.ds(j * 16, 16)] = jnp.zeros((16,), jnp.float32)
    return 0
  lax.fori_loop(0, nrows * k, body, 0)


def _load_slab(src, slab, wid):
  pltpu.sync_copy(src.at[pl.ds(wid * PW, PW)], slab)


def _fill(dst, slab, j):
  def fb(k, _):
    dst[pl.ds(k * 16, 16)] = slab[pl.ds(j * C + k * 16, 16)]
    return 0
  lax.fori_loop(0, C // 16, fb, 0)


@functools.partial(
    pl.kernel,
    out_type=(jax.ShapeDtypeStruct((NC * NV_PAD,), jnp.float32),
              jax.ShapeDtypeStruct((NC * NE_PAD,), jnp.float32)),
    mesh=_mesh(),
    scratch_types=[
        pltpu.VMEM((SLAB,), jnp.int32),
        pltpu.VMEM((SLAB,), jnp.int32),
        pltpu.VMEM((C,), jnp.int32),
        pltpu.VMEM((C,), jnp.int32),
        pltpu.VMEM((C,), jnp.int32),
        pltpu.VMEM((C,), jnp.int32),
        pltpu.VMEM((C,), jnp.float32),
        pltpu.VMEM((NV_PAD // NS,), jnp.float32),
        pltpu.VMEM_SHARED((NV_PAD,), jnp.float32),
        pltpu.VMEM_SHARED((NE_PAD,), jnp.float32),
        pltpu.SemaphoreType.DMA,
        pltpu.SemaphoreType.DMA,
        pltpu.SemaphoreType.DMA,
        pltpu.SemaphoreType.DMA,
    ],
)
def _hist(nidx, hidx, dv_out, de_out,
          nl, hl, in0, in1, ih0, ih1, ones_v, zeros_v, dv_s, de_s,
          sn0, sn1, sh0, sh1):
  c = lax.axis_index("c")
  s = lax.axis_index("s")
  wid = s * NC + c

  _load_slab(nidx, nl, wid)
  _load_slab(hidx, hl, wid)
  _zero_1d(zeros_v, NV_PAD // NS)

  def ob(i, _):
    ones_v[pl.ds(i * 16, 16)] = jnp.ones((16,), jnp.float32)
    return 0
  lax.fori_loop(0, C // 16, ob, 0)

  vslice = NV_PAD // NS
  eslice = NE_PAD // NS
  pltpu.sync_copy(zeros_v, dv_s.at[pl.ds(s * vslice, vslice)])
  pltpu.sync_copy(zeros_v.at[pl.ds(0, eslice)], de_s.at[pl.ds(s * eslice, eslice)])
  plsc.subcore_barrier()

  def slot_step(j, me_in, me_ih, me_sn, me_sh):
    @pl.when(j >= 2)
    def _():
      pltpu.make_async_copy(ones_v, dv_s.at[me_in], me_sn).wait()
      pltpu.make_async_copy(ones_v, de_s.at[me_ih], me_sh).wait()
    _fill(me_in, nl, j)
    _fill(me_ih, hl, j)
    pltpu.async_copy(ones_v, dv_s.at[me_in], me_sn, add=True)
    pltpu.async_copy(ones_v, de_s.at[me_ih], me_sh, add=True)

  def body(j, _):
    @pl.when(j % 2 == 0)
    def _():
      slot_step(j, in0, ih0, sn0, sh0)
    @pl.when(j % 2 == 1)
    def _():
      slot_step(j, in1, ih1, sn1, sh1)
    return 0
  lax.fori_loop(0, NCH, body, 0)

  pltpu.make_async_copy(ones_v, dv_s.at[in0], sn0).wait()
  pltpu.make_async_copy(ones_v, de_s.at[ih0], sh0).wait()
  pltpu.make_async_copy(ones_v, dv_s.at[in1], sn1).wait()
  pltpu.make_async_copy(ones_v, de_s.at[ih1], sh1).wait()
  plsc.subcore_barrier()

  pltpu.sync_copy(dv_s.at[pl.ds(s * vslice, vslice)], zeros_v)
  pltpu.sync_copy(zeros_v, dv_out.at[pl.ds(c * NV_PAD + s * vslice, vslice)])
  pltpu.sync_copy(de_s.at[pl.ds(s * eslice, eslice)],
                  zeros_v.at[pl.ds(0, eslice)])
  pltpu.sync_copy(zeros_v.at[pl.ds(0, eslice)],
                  de_out.at[pl.ds(c * NE_PAD + s * eslice, eslice)])


def _proj_body(x_ref, w_ref, b_ref, dvp_ref, dep_ref, hs_ref, dvi_ref, dei_ref):
  dv = dvp_ref[0] + dvp_ref[1]
  dvi = jnp.where(dv > 0, lax.rsqrt(dv), 0.0)
  de = dep_ref[0] + dep_ref[1]
  dei = jnp.where(de > 0, 1.0 / de, 0.0)
  h = lax.dot_general(x_ref[...], w_ref[...], (((1,), (1,)), ((), ())),
                      preferred_element_type=jnp.float32,
                      precision=lax.Precision.HIGHEST)
  hs_ref[...] = (h + b_ref[...]) * dvi
  dvi_ref[...] = dvi
  dei_ref[...] = dei


def _proj(x, w, b2, dvp, dep):
  return pl.pallas_call(
      _proj_body,
      out_shape=(jax.ShapeDtypeStruct((NV, D), jnp.float32),
                 jax.ShapeDtypeStruct((NV, 1), jnp.float32),
                 jax.ShapeDtypeStruct((NE, 1), jnp.float32)),
  )(x, w, b2, dvp, dep)


def _make_agg(acc_rows):
  @functools.partial(
      pl.kernel,
      out_type=jax.ShapeDtypeStruct((NC, acc_rows, D), jnp.float32),
      mesh=_mesh(),
      scratch_types=[
          pltpu.VMEM((SLAB,), jnp.int32),
          pltpu.VMEM((SLAB,), jnp.int32),
          pltpu.VMEM((C,), jnp.int32),
          pltpu.VMEM((C,), jnp.int32),
          pltpu.VMEM((C,), jnp.int32),
          pltpu.VMEM((C,), jnp.int32),
          pltpu.VMEM((C, D), jnp.float32),
          pltpu.VMEM((C, D), jnp.float32),
          pltpu.VMEM_SHARED((acc_rows, D), jnp.float32),
          pltpu.SemaphoreType.DMA,
          pltpu.SemaphoreType.DMA,
          pltpu.SemaphoreType.DMA,
          pltpu.SemaphoreType.DMA,
      ],
  )
  def agg(table, gidx, sidx, out,
          gl, sl, g0, g1, s0, s1, rows0, rows1, acc_s, gs0, gs1, ss0, ss1):
    c = lax.axis_index("c")
    s = lax.axis_index("s")
    wid = s * NC + c

    _load_slab(gidx, gl, wid)
    _load_slab(sidx, sl, wid)
    _fill(g0, gl, 0)
    _fill(s0, sl, 0)
    pltpu.async_copy(table.at[g0], rows0, gs0)

    _zero_2d(rows1, C, D)
    rows_per_tile = acc_rows // NS

    def zb(i, _):
      pltpu.sync_copy(rows1, acc_s.at[pl.ds(s * rows_per_tile + i * C, C)])
      return 0
    lax.fori_loop(0, rows_per_tile // C, zb, 0)
    plsc.subcore_barrier()

    def slot_step(j, me_rows, me_gs, me_ss, me_g, me_s,
                  ot_rows, ot_gs, ot_ss, ot_g, ot_s):
      @pl.when(j + 1 < NCH)
      def _():
        @pl.when(j >= 1)
        def _():
          pltpu.make_async_copy(ot_rows, acc_s.at[ot_s], ot_ss).wait()
        _fill(ot_g, gl, j + 1)
        _fill(ot_s, sl, j + 1)
        pltpu.async_copy(table.at[ot_g], ot_rows, ot_gs)
      pltpu.make_async_copy(table.at[me_g], me_rows, me_gs).wait()
      pltpu.async_copy(me_rows, acc_s.at[me_s], me_ss, add=True)

    def body(j, _):
      @pl.when(j % 2 == 0)
      def _():
        slot_step(j, rows0, gs0, ss0, g0, s0, rows1, gs1, ss1, g1, s1)
      @pl.when(j % 2 == 1)
      def _():
        slot_step(j, rows1, gs1, ss1, g1, s1, rows0, gs0, ss0, g0, s0)
      return 0
    lax.fori_loop(0, NCH, body, 0)

    pltpu.make_async_copy(rows0, acc_s.at[s0], ss0).wait()
    pltpu.make_async_copy(rows1, acc_s.at[s1], ss1).wait()
    plsc.subcore_barrier()

    def db(i, _):
      off = s * rows_per_tile + i * C
      pltpu.sync_copy(acc_s.at[pl.ds(off, C)], rows0)
      pltpu.sync_copy(rows0, out.at[c, pl.ds(off, C)])
      return 0
    lax.fori_loop(0, rows_per_tile // C, db, 0)

  return agg


_hedge_agg = _make_agg(NE_PAD)
_node_agg = _make_agg(NV_PAD)


def _ye_combine_body(yep_ref, dei_ref, ye_ref):
  ye = yep_ref[0, :NE, :] + yep_ref[1, :NE, :]
  ye_ref[...] = ye * dei_ref[...]


def _ye_combine(yep, dei):
  return pl.pallas_call(
      _ye_combine_body,
      out_shape=jax.ShapeDtypeStruct((NE, D), jnp.float32),
  )(yep, dei)


def _z_combine_body(zp_ref, dvi_ref, z_ref):
  z = zp_ref[0, :NV, :] + zp_ref[1, :NV, :]
  z_ref[...] = jnp.maximum(z * dvi_ref[...], 0.0)


def _z_combine(zp, dvi):
  return pl.pallas_call(
      _z_combine_body,
      out_shape=jax.ShapeDtypeStruct((NV, D), jnp.float32),
  )(zp, dvi)


def kernel(X, node_idx, hedge_idx, W, b):
  nidx = node_idx.astype(jnp.int32)
  hidx = hedge_idx.astype(jnp.int32)
  dvp, dep = _hist(nidx, hidx)
  dvp = dvp.reshape(NC, NV_PAD)[:, :NV, None]
  dep = dep.reshape(NC, NE_PAD)[:, :NE, None]
  hs, dvi, dei = _proj(X, W, b.reshape(1, D), dvp, dep)
  yep = _hedge_agg(hs, nidx, hidx)
  ye = _ye_combine(yep, dei)
  zp = _node_agg(ye, hidx, nidx)
  return _z_combine(zp, dvi)
